# Initial kernel scaffold; baseline (speedup 1.0000x reference)
import jax
import jax.numpy as jnp
from jax.experimental import pallas as pl


def kernel(text_features, projected_layer_features, W1_w, W1_b, Wc_w, Wc_b, Wi_w, Wi_b, Wf_w, Wf_b, bc, bi, bf, score_q_w, score_k_w, ln_g, ln_b):
    raise NotImplementedError("write your pallas kernel here")


# speedup vs baseline: 2.0996x; 2.0996x over previous
import functools

import jax
import jax.numpy as jnp
from jax import lax
from jax.experimental import pallas as pl
from jax.experimental.pallas import tpu as pltpu
from jax.experimental.pallas import tpu_sc as plsc

_T, _D, _L, _P = 2048, 768, 24, 1024
_RD = 192
_FOCUS = 22
_KCAP = 64
_FOCUS_MIN = 32
_FINAL_K = 64
_REST = 64
_THRESH = 2.0
_NEG = float("-inf")


def _ln_rows(x, g, b, eps=1e-5):
    mu = jnp.mean(x, axis=-1, keepdims=True)
    var = jnp.mean((x - mu) ** 2, axis=-1, keepdims=True)
    return (x - mu) / jnp.sqrt(var + eps) * g + b


def _dot_t(a, b):
    return lax.dot_general(a, b, (((1,), (1,)), ((), ())),
                           preferred_element_type=jnp.float32)


def _score_body(text_ref, pl_ref, w1_ref, w1b_ref, wc_ref, wcb_ref, wi_ref,
                wib_ref, wf_ref, wfb_ref, bc_ref, bi_ref, bf_ref, qw_ref,
                kw_ref, g_ref, b_ref, sel_ref, y_ref, q_ref, zm_ref, tv_ref,
                ti_ref):
    phase = pl.program_id(0)
    li = pl.program_id(1)

    @pl.when(phase == 0)
    def _means():
        x = pl_ref[0]
        y_ref[pl.ds(li, 1), :] = jnp.mean(x, axis=0, keepdims=True)

    @pl.when((phase == 0) & (li == _L - 1))
    def _recurrence():
        tm = jnp.mean(text_ref[...], axis=0, keepdims=True)
        tg = _ln_rows(tm, 1.0, 0.0)
        g = g_ref[0:1, :]
        b = b_ref[0:1, :]
        c = jnp.zeros((1, _D), jnp.float32)
        for i in range(_L - 1):
            y = y_ref[i:i + 1, :]
            cn = jax.nn.sigmoid(c)
            comb = jnp.concatenate([cn, y, tg], axis=1)
            s = jnp.maximum(_dot_t(comb, w1_ref[...]) + w1b_ref[0:1, :], 0.0)
            ct = jnp.tanh(_dot_t(s, wc_ref[...]) + wcb_ref[0:1, :]
                          + bc_ref[0:1, :])
            ig = jax.nn.sigmoid(_dot_t(s, wi_ref[...]) + wib_ref[0:1, :]
                                + bi_ref[0:1, :])
            fg = jax.nn.sigmoid(_dot_t(s, wf_ref[...]) + wfb_ref[0:1, :]
                                + bf_ref[0:1, :])
            c = fg * c + ig * ct
        q = _ln_rows(_dot_t(c, qw_ref[...]), g, b)
        q_ref[0:1, :] = q

    @pl.when(phase == 1)
    def _scores():
        x = pl_ref[0]
        kk = _ln_rows(_dot_t(x, kw_ref[...]), g_ref[0:1, :], b_ref[0:1, :])
        sc = jnp.sum(kk * q_ref[0:1, :], axis=1, keepdims=True)
        m = jnp.mean(sc)
        sd = jnp.sqrt(jnp.mean((sc - m) ** 2))
        z = (sc - m) / (sd + 1e-6)
        zm_ref[pl.ds(li, 1), :] = z.reshape(1, _P)

    @pl.when((phase == 1) & (li == _L - 1))
    def _select():
        lane_p = lax.broadcasted_iota(jnp.int32, (_L, _P), 1)
        for k in range(_KCAP):
            zm = zm_ref[...]
            mx = jnp.max(zm, axis=1, keepdims=True)
            idx = jnp.min(jnp.where(zm == mx, lane_p, _P), axis=1,
                          keepdims=True)
            tv_ref[:, k:k + 1] = mx
            ti_ref[:, k:k + 1] = idx
            zm_ref[...] = jnp.where(lane_p == idx, _NEG, zm)

        i_col = lax.broadcasted_iota(jnp.int32, (_L, 1), 0)
        conf = tv_ref[:, 0:1]
        mask = (conf > _THRESH) & (i_col != _FOCUS)
        cmax = jnp.max(jnp.where(mask, conf, _NEG), axis=0, keepdims=True)
        e = jnp.where(mask, jnp.exp(conf - cmax), 0.0)
        s_sum = jnp.sum(e, axis=0, keepdims=True)
        w = e / jnp.where(s_sum > 0.0, s_sum, 1.0)
        alloc = jnp.floor(w * float(_REST)).astype(jnp.int32)
        rem = jnp.int32(_REST) - jnp.sum(alloc, axis=0, keepdims=True)
        rank = jnp.zeros((_L, 1), jnp.int32)
        for j in range(_L):
            wj = w[j:j + 1, 0:1]
            rank = rank + jnp.where(wj > w, 1, 0) \
                        + jnp.where((wj == w) & (j < i_col), 1, 0)
        alloc = alloc + jnp.where(mask & (rank < rem), 1, 0)
        kpl = jnp.where(mask, jnp.minimum(alloc, _P), 0)
        kpl = jnp.where(i_col == _FOCUS, _KCAP, kpl)

        col = lax.broadcasted_iota(jnp.int32, (_L, _KCAP), 1)
        row = lax.broadcasted_iota(jnp.int32, (_L, _KCAP), 0)
        cand = jnp.where(col < kpl, tv_ref[...], _NEG)
        rem_sc = jnp.where((row == _FOCUS) & (col < _FOCUS_MIN), _NEG, cand)
        flat = row * _KCAP + col

        sel_ref[0:1, 0:_FOCUS_MIN] = (
            jnp.int32(_FOCUS * _P) + ti_ref[_FOCUS:_FOCUS + 1, 0:_FOCUS_MIN])
        for k in range(_FINAL_K - _FOCUS_MIN):
            mx = jnp.max(rem_sc, axis=(0, 1), keepdims=True)
            fidx = jnp.min(jnp.where(rem_sc == mx, flat, _L * _KCAP),
                           axis=(0, 1), keepdims=True)
            onehot = flat == fidx
            pid = jnp.sum(jnp.where(onehot, ti_ref[...], 0), axis=(0, 1),
                          keepdims=True)
            lid = fidx // _KCAP
            sel_ref[0:1, _FOCUS_MIN + k:_FOCUS_MIN + k + 1] = (
                lid * _P + pid)
            rem_sc = jnp.where(onehot, _NEG, rem_sc)


def _compute_sel(text_features, projected_layer_features, W1_w, W1_b, Wc_w,
                 Wc_b, Wi_w, Wi_b, Wf_w, Wf_b, bc, bi, bf, score_q_w,
                 score_k_w, ln_g, ln_b):
    def c2(p, l):
        return (0, 0)
    grid = (2, _L)
    return pl.pallas_call(
        _score_body,
        grid=grid,
        in_specs=[
            pl.BlockSpec((_T, _D), c2),
            pl.BlockSpec((1, _P, _D), lambda p, l: (l, 0, 0)),
            pl.BlockSpec((_RD, 3 * _D), c2),
            pl.BlockSpec((1, _RD), c2),
            pl.BlockSpec((_D, _RD), c2),
            pl.BlockSpec((1, _D), c2),
            pl.BlockSpec((_D, _RD), c2),
            pl.BlockSpec((1, _D), c2),
            pl.BlockSpec((_D, _RD), c2),
            pl.BlockSpec((1, _D), c2),
            pl.BlockSpec((1, _D), c2),
            pl.BlockSpec((1, _D), c2),
            pl.BlockSpec((1, _D), c2),
            pl.BlockSpec((_D, _D), c2),
            pl.BlockSpec((_D, _D), c2),
            pl.BlockSpec((1, _D), c2),
            pl.BlockSpec((1, _D), c2),
        ],
        out_specs=pl.BlockSpec((8, _FINAL_K), c2),
        out_shape=jax.ShapeDtypeStruct((8, _FINAL_K), jnp.int32),
        scratch_shapes=[
            pltpu.VMEM((_L, _D), jnp.float32),
            pltpu.VMEM((8, _D), jnp.float32),
            pltpu.VMEM((_L, _P), jnp.float32),
            pltpu.VMEM((_L, _KCAP), jnp.float32),
            pltpu.VMEM((_L, _KCAP), jnp.int32),
        ],
    )(text_features, projected_layer_features, W1_w, W1_b.reshape(1, _RD),
      Wc_w, Wc_b.reshape(1, _D), Wi_w, Wi_b.reshape(1, _D), Wf_w,
      Wf_b.reshape(1, _D), bc.reshape(1, _D), bi.reshape(1, _D),
      bf.reshape(1, _D), score_q_w, score_k_w, ln_g.reshape(1, _D),
      ln_b.reshape(1, _D))


_NW_ROWS = 8


def _gather_rows(table, idx):
    info = plsc.get_sparse_core_info()
    nc = info.num_cores
    mesh = plsc.VectorSubcoreMesh(core_axis_name="c", subcore_axis_name="s")

    @functools.partial(
        pl.kernel,
        mesh=mesh,
        out_type=jax.ShapeDtypeStruct((_FINAL_K, _D), jnp.float32),
        scratch_types=[
            pltpu.VMEM((_NW_ROWS,), jnp.int32),
            pltpu.VMEM((_NW_ROWS, _D), jnp.float32),
            pltpu.SemaphoreType.DMA,
        ],
    )
    def gather_k(table_hbm, idx_hbm, out_hbm, idx_v, rows_v, sem):
        wid = lax.axis_index("s") * nc + lax.axis_index("c")

        @pl.when(wid < _FINAL_K // _NW_ROWS)
        def _():
            base = wid * _NW_ROWS
            pltpu.sync_copy(idx_hbm.at[pl.ds(base, _NW_ROWS)], idx_v)
            pltpu.async_copy(table_hbm.at[idx_v], rows_v, sem).wait()
            pltpu.sync_copy(rows_v, out_hbm.at[pl.ds(base, _NW_ROWS)])

    return gather_k(table, idx)


def kernel(text_features, projected_layer_features, W1_w, W1_b, Wc_w, Wc_b,
           Wi_w, Wi_b, Wf_w, Wf_b, bc, bi, bf, score_q_w, score_k_w, ln_g,
           ln_b):
    sel = _compute_sel(text_features, projected_layer_features, W1_w, W1_b,
                       Wc_w, Wc_b, Wi_w, Wi_b, Wf_w, Wf_b, bc, bi, bf,
                       score_q_w, score_k_w, ln_g, ln_b)
    idx = sel[0]
    table = projected_layer_features.reshape(_L * _P, _D)
    return _gather_rows(table, idx)


# speedup vs baseline: 2.2218x; 1.0582x over previous
import functools

import jax
import jax.numpy as jnp
from jax import lax
from jax.experimental import pallas as pl
from jax.experimental.pallas import tpu as pltpu
from jax.experimental.pallas import tpu_sc as plsc

_T, _D, _L, _P = 2048, 768, 24, 1024
_RD = 192
_FOCUS = 22
_KCAP = 64
_FOCUS_MIN = 32
_FINAL_K = 64
_REST = 64
_THRESH = 2.0
_NEG = float("-inf")


def _ln_rows(x, g, b, eps=1e-5):
    mu = jnp.mean(x, axis=-1, keepdims=True)
    var = jnp.mean((x - mu) ** 2, axis=-1, keepdims=True)
    return (x - mu) / jnp.sqrt(var + eps) * g + b


def _dot_t(a, b):
    return lax.dot_general(a, b, (((1,), (1,)), ((), ())),
                           preferred_element_type=jnp.float32)


def _score_body(text_ref, pl_ref, w1_ref, w1b_ref, wcif_ref, wcifb_ref,
                qw_ref, kw_ref, g_ref, b_ref, sel_ref, y_ref, q_ref, zm_ref,
                tv_ref, ti_ref):
    i = pl.program_id(0)
    li = jnp.where(i < _L - 1, i, i - (_L - 1))

    @pl.when(i < _L - 1)
    def _means():
        x = pl_ref[0]
        y_ref[pl.ds(li, 1), :] = jnp.mean(x, axis=0, keepdims=True)

    @pl.when(i == _L - 2)
    def _recurrence():
        tm = jnp.mean(text_ref[...], axis=0, keepdims=True)
        tg = _ln_rows(tm, 1.0, 0.0)
        g = g_ref[0:1, :]
        b = b_ref[0:1, :]
        c = jnp.zeros((1, _D), jnp.float32)
        for it in range(_L - 1):
            y = y_ref[it:it + 1, :]
            cn = jax.nn.sigmoid(c)
            comb = jnp.concatenate([cn, y, tg], axis=1)
            s = jnp.maximum(_dot_t(comb, w1_ref[...]) + w1b_ref[0:1, :], 0.0)
            u = _dot_t(s, wcif_ref[...]) + wcifb_ref[0:1, :]
            ct = jnp.tanh(u[:, 0:_D])
            ig = jax.nn.sigmoid(u[:, _D:2 * _D])
            fg = jax.nn.sigmoid(u[:, 2 * _D:3 * _D])
            c = fg * c + ig * ct
        q = _ln_rows(_dot_t(c, qw_ref[...]), g, b)
        q_ref[0:1, :] = q

    @pl.when(i >= _L - 1)
    def _scores():
        x = pl_ref[0]
        kkr = _dot_t(x, kw_ref[...])
        mu = jnp.mean(kkr, axis=1, keepdims=True)
        var = jnp.mean((kkr - mu) ** 2, axis=1, keepdims=True)
        inv = 1.0 / jnp.sqrt(var + 1e-5)
        kk = (kkr - mu) * inv * g_ref[0:1, :] + b_ref[0:1, :]
        sc = jnp.sum(kk * q_ref[0:1, :], axis=1, keepdims=True)
        m = jnp.mean(sc)
        sd = jnp.sqrt(jnp.mean((sc - m) ** 2))
        z = (sc - m) / (sd + 1e-6)
        zm_ref[pl.ds(li, 1), :] = z.reshape(1, _P)

    @pl.when(i == 2 * _L - 2)
    def _select():
        lane_p = lax.broadcasted_iota(jnp.int32, (_L, _P), 1)
        zm = zm_ref[...]
        for k in range(_KCAP):
            mx = jnp.max(zm, axis=1, keepdims=True)
            idx = jnp.min(jnp.where(zm == mx, lane_p, _P), axis=1,
                          keepdims=True)
            tv_ref[:, k:k + 1] = mx
            ti_ref[:, k:k + 1] = idx
            zm = jnp.where(lane_p == idx, _NEG, zm)

        i_col = lax.broadcasted_iota(jnp.int32, (_L, 1), 0)
        conf = tv_ref[:, 0:1]
        mask = (conf > _THRESH) & (i_col != _FOCUS)
        cmax = jnp.max(jnp.where(mask, conf, _NEG), axis=0, keepdims=True)
        e = jnp.where(mask, jnp.exp(conf - cmax), 0.0)
        s_sum = jnp.sum(e, axis=0, keepdims=True)
        w = e / jnp.where(s_sum > 0.0, s_sum, 1.0)
        alloc = jnp.floor(w * float(_REST)).astype(jnp.int32)
        rem = jnp.int32(_REST) - jnp.sum(alloc, axis=0, keepdims=True)
        rank = jnp.zeros((_L, 1), jnp.int32)
        for j in range(_L):
            wj = w[j:j + 1, 0:1]
            rank = rank + jnp.where(wj > w, 1, 0) \
                        + jnp.where((wj == w) & (j < i_col), 1, 0)
        alloc = alloc + jnp.where(mask & (rank < rem), 1, 0)
        kpl = jnp.where(mask, jnp.minimum(alloc, _P), 0)
        kpl = jnp.where(i_col == _FOCUS, _KCAP, kpl)

        col = lax.broadcasted_iota(jnp.int32, (_L, _KCAP), 1)
        row = lax.broadcasted_iota(jnp.int32, (_L, _KCAP), 0)
        cand = jnp.where(col < kpl, tv_ref[...], _NEG)
        rem_sc = jnp.where((row == _FOCUS) & (col < _FOCUS_MIN), _NEG, cand)
        flat = row * _KCAP + col
        tiv = ti_ref[...]

        sel_ref[0:1, 0:_FOCUS_MIN] = (
            jnp.int32(_FOCUS * _P) + ti_ref[_FOCUS:_FOCUS + 1, 0:_FOCUS_MIN])
        for k in range(_FINAL_K - _FOCUS_MIN):
            mx = jnp.max(rem_sc, axis=(0, 1), keepdims=True)
            fidx = jnp.min(jnp.where(rem_sc == mx, flat, _L * _KCAP),
                           axis=(0, 1), keepdims=True)
            onehot = flat == fidx
            pid = jnp.sum(jnp.where(onehot, tiv, 0), axis=(0, 1),
                          keepdims=True)
            lid = fidx // _KCAP
            sel_ref[0:1, _FOCUS_MIN + k:_FOCUS_MIN + k + 1] = (
                lid * _P + pid)
            rem_sc = jnp.where(onehot, _NEG, rem_sc)


def _compute_sel(text_features, projected_layer_features, W1_w, W1_b, Wc_w,
                 Wc_b, Wi_w, Wi_b, Wf_w, Wf_b, bc, bi, bf, score_q_w,
                 score_k_w, ln_g, ln_b):
    def c0(i):
        return (0, 0)
    wcif = jnp.concatenate([Wc_w, Wi_w, Wf_w], axis=0)
    wcifb = jnp.concatenate([Wc_b + bc, Wi_b + bi, Wf_b + bf])
    return pl.pallas_call(
        _score_body,
        grid=(2 * _L - 1,),
        in_specs=[
            pl.BlockSpec((_T, _D), c0),
            pl.BlockSpec((1, _P, _D),
                         lambda i: (jnp.where(i < _L - 1, i, i - (_L - 1)),
                                    0, 0)),
            pl.BlockSpec((_RD, 3 * _D), c0),
            pl.BlockSpec((1, _RD), c0),
            pl.BlockSpec((3 * _D, _RD), c0),
            pl.BlockSpec((1, 3 * _D), c0),
            pl.BlockSpec((_D, _D), c0),
            pl.BlockSpec((_D, _D), c0),
            pl.BlockSpec((1, _D), c0),
            pl.BlockSpec((1, _D), c0),
        ],
        out_specs=pl.BlockSpec((8, _FINAL_K), c0),
        out_shape=jax.ShapeDtypeStruct((8, _FINAL_K), jnp.int32),
        scratch_shapes=[
            pltpu.VMEM((_L, _D), jnp.float32),
            pltpu.VMEM((8, _D), jnp.float32),
            pltpu.VMEM((_L, _P), jnp.float32),
            pltpu.VMEM((_L, _KCAP), jnp.float32),
            pltpu.VMEM((_L, _KCAP), jnp.int32),
        ],
    )(text_features, projected_layer_features, W1_w, W1_b.reshape(1, _RD),
      wcif, wcifb.reshape(1, 3 * _D), score_q_w, score_k_w,
      ln_g.reshape(1, _D), ln_b.reshape(1, _D))


_NW_ROWS = 8


def _gather_rows(table, idx):
    info = plsc.get_sparse_core_info()
    nc = info.num_cores
    mesh = plsc.VectorSubcoreMesh(core_axis_name="c", subcore_axis_name="s")

    @functools.partial(
        pl.kernel,
        mesh=mesh,
        out_type=jax.ShapeDtypeStruct((_FINAL_K, _D), jnp.float32),
        scratch_types=[
            pltpu.VMEM((_NW_ROWS,), jnp.int32),
            pltpu.VMEM((_NW_ROWS, _D), jnp.float32),
            pltpu.SemaphoreType.DMA,
        ],
    )
    def gather_k(table_hbm, idx_hbm, out_hbm, idx_v, rows_v, sem):
        wid = lax.axis_index("s") * nc + lax.axis_index("c")

        @pl.when(wid < _FINAL_K // _NW_ROWS)
        def _():
            base = wid * _NW_ROWS
            pltpu.sync_copy(idx_hbm.at[pl.ds(base, _NW_ROWS)], idx_v)
            pltpu.async_copy(table_hbm.at[idx_v], rows_v, sem).wait()
            pltpu.sync_copy(rows_v, out_hbm.at[pl.ds(base, _NW_ROWS)])

    return gather_k(table, idx)


def kernel(text_features, projected_layer_features, W1_w, W1_b, Wc_w, Wc_b,
           Wi_w, Wi_b, Wf_w, Wf_b, bc, bi, bf, score_q_w, score_k_w, ln_g,
           ln_b):
    sel = _compute_sel(text_features, projected_layer_features, W1_w, W1_b,
                       Wc_w, Wc_b, Wi_w, Wi_b, Wf_w, Wf_b, bc, bi, bf,
                       score_q_w, score_k_w, ln_g, ln_b)
    idx = sel[0]
    table = projected_layer_features.reshape(_L * _P, _D)
    return _gather_rows(table, idx)


# speedup vs baseline: 2.3766x; 1.0697x over previous
import functools

import jax
import jax.numpy as jnp
from jax import lax
from jax.experimental import pallas as pl
from jax.experimental.pallas import tpu as pltpu
from jax.experimental.pallas import tpu_sc as plsc

_T, _D, _L, _P = 2048, 768, 24, 1024
_RD = 192
_FOCUS = 22
_KCAP = 64
_FOCUS_MIN = 32
_FINAL_K = 64
_REST = 64
_THRESH = 2.0
_NEG = float("-inf")


def _ln_rows(x, g, b, eps=1e-5):
    mu = jnp.mean(x, axis=-1, keepdims=True)
    var = jnp.mean((x - mu) ** 2, axis=-1, keepdims=True)
    return (x - mu) / jnp.sqrt(var + eps) * g + b


def _dot_t(a, b):
    return lax.dot_general(a, b, (((1,), (1,)), ((), ())),
                           preferred_element_type=jnp.float32)


def _score_body(text_ref, pl_ref, w1_ref, w1b_ref, wcif_ref, wcifb_ref,
                qw_ref, kw_ref, g_ref, b_ref, sel_ref, q_ref, zm_ref,
                tv_ref, ti_ref):
    i = pl.program_id(0)
    li = jnp.where(i < _L - 1, i, i - (_L - 1))

    @pl.when(i == 0)
    def _init():
        tm = jnp.mean(text_ref[...], axis=0, keepdims=True)
        q_ref[2:3, :] = _ln_rows(tm, 1.0, 0.0)
        q_ref[1:2, :] = jnp.zeros((1, _D), jnp.float32)

    @pl.when(i < _L - 1)
    def _rec_step():
        x = pl_ref[0]
        ym = jnp.mean(x, axis=0, keepdims=True)
        tg = q_ref[2:3, :]
        c = q_ref[1:2, :]
        cn = jax.nn.sigmoid(c)
        comb = jnp.concatenate([cn, ym, tg], axis=1)
        s = jnp.maximum(_dot_t(comb, w1_ref[...]) + w1b_ref[0:1, :], 0.0)
        u = _dot_t(s, wcif_ref[...]) + wcifb_ref[0:1, :]
        ct = jnp.tanh(u[:, 0:_D])
        ig = jax.nn.sigmoid(u[:, _D:2 * _D])
        fg = jax.nn.sigmoid(u[:, 2 * _D:3 * _D])
        c = fg * c + ig * ct
        q_ref[1:2, :] = c

        @pl.when(i == _L - 2)
        def _finalize_q():
            q_ref[0:1, :] = _ln_rows(_dot_t(c, qw_ref[...]), g_ref[0:1, :],
                                     b_ref[0:1, :])

    @pl.when(i >= _L - 1)
    def _scores():
        x = pl_ref[0]
        kkr = _dot_t(x, kw_ref[...])
        mu = jnp.mean(kkr, axis=1, keepdims=True)
        d = kkr - mu
        var = jnp.mean(d * d, axis=1, keepdims=True)
        inv = 1.0 / jnp.sqrt(var + 1e-5)
        kk = d * inv * g_ref[0:1, :] + b_ref[0:1, :]
        sc = jnp.sum(kk * q_ref[0:1, :], axis=1, keepdims=True)
        m = jnp.mean(sc)
        sd = jnp.sqrt(jnp.mean((sc - m) ** 2))
        z = (sc - m) / (sd + 1e-6)
        zm_ref[pl.ds(li, 1), :] = z.reshape(1, _P)

    @pl.when(i == 2 * _L - 2)
    def _select():
        lane_p = lax.broadcasted_iota(jnp.int32, (_L, _P), 1)
        zm = zm_ref[...]
        for k in range(_KCAP):
            mx = jnp.max(zm, axis=1, keepdims=True)
            idx = jnp.min(jnp.where(zm == mx, lane_p, _P), axis=1,
                          keepdims=True)
            tv_ref[:, k:k + 1] = mx
            ti_ref[:, k:k + 1] = idx
            zm = jnp.where(lane_p == idx, _NEG, zm)

        i_col = lax.broadcasted_iota(jnp.int32, (_L, 1), 0)
        conf = tv_ref[:, 0:1]
        mask = (conf > _THRESH) & (i_col != _FOCUS)
        cmax = jnp.max(jnp.where(mask, conf, _NEG), axis=0, keepdims=True)
        e = jnp.where(mask, jnp.exp(conf - cmax), 0.0)
        s_sum = jnp.sum(e, axis=0, keepdims=True)
        w = e / jnp.where(s_sum > 0.0, s_sum, 1.0)
        alloc = jnp.floor(w * float(_REST)).astype(jnp.int32)
        rem = jnp.int32(_REST) - jnp.sum(alloc, axis=0, keepdims=True)
        rank = jnp.zeros((_L, 1), jnp.int32)
        for j in range(_L):
            wj = w[j:j + 1, 0:1]
            rank = rank + jnp.where(wj > w, 1, 0) \
                        + jnp.where((wj == w) & (j < i_col), 1, 0)
        alloc = alloc + jnp.where(mask & (rank < rem), 1, 0)
        kpl = jnp.where(mask, jnp.minimum(alloc, _P), 0)
        kpl = jnp.where(i_col == _FOCUS, _KCAP, kpl)

        col = lax.broadcasted_iota(jnp.int32, (_L, _KCAP), 1)
        row = lax.broadcasted_iota(jnp.int32, (_L, _KCAP), 0)
        cand = jnp.where(col < kpl, tv_ref[...], _NEG)
        rem_sc = jnp.where((row == _FOCUS) & (col < _FOCUS_MIN), _NEG, cand)
        flat = row * _KCAP + col
        tiv = ti_ref[...]

        sel_ref[0:1, 0:_FOCUS_MIN] = (
            jnp.int32(_FOCUS * _P) + ti_ref[_FOCUS:_FOCUS + 1, 0:_FOCUS_MIN])
        for k in range(_FINAL_K - _FOCUS_MIN):
            mx = jnp.max(rem_sc, axis=(0, 1), keepdims=True)
            fidx = jnp.min(jnp.where(rem_sc == mx, flat, _L * _KCAP),
                           axis=(0, 1), keepdims=True)
            onehot = flat == fidx
            pid = jnp.sum(jnp.where(onehot, tiv, 0), axis=(0, 1),
                          keepdims=True)
            lid = fidx // _KCAP
            sel_ref[0:1, _FOCUS_MIN + k:_FOCUS_MIN + k + 1] = (
                lid * _P + pid)
            rem_sc = jnp.where(onehot, _NEG, rem_sc)


def _compute_sel(text_features, projected_layer_features, W1_w, W1_b, Wc_w,
                 Wc_b, Wi_w, Wi_b, Wf_w, Wf_b, bc, bi, bf, score_q_w,
                 score_k_w, ln_g, ln_b):
    def c0(i):
        return (0, 0)
    wcif = jnp.concatenate([Wc_w, Wi_w, Wf_w], axis=0)
    wcifb = jnp.concatenate([Wc_b + bc, Wi_b + bi, Wf_b + bf])
    return pl.pallas_call(
        _score_body,
        grid=(2 * _L - 1,),
        in_specs=[
            pl.BlockSpec((_T, _D), c0),
            pl.BlockSpec((1, _P, _D),
                         lambda i: (jnp.where(i < _L - 1, i, i - (_L - 1)),
                                    0, 0)),
            pl.BlockSpec((_RD, 3 * _D), c0),
            pl.BlockSpec((1, _RD), c0),
            pl.BlockSpec((3 * _D, _RD), c0),
            pl.BlockSpec((1, 3 * _D), c0),
            pl.BlockSpec((_D, _D), c0),
            pl.BlockSpec((_D, _D), c0),
            pl.BlockSpec((1, _D), c0),
            pl.BlockSpec((1, _D), c0),
        ],
        out_specs=pl.BlockSpec((8, _FINAL_K), c0),
        out_shape=jax.ShapeDtypeStruct((8, _FINAL_K), jnp.int32),
        scratch_shapes=[
            pltpu.VMEM((8, _D), jnp.float32),
            pltpu.VMEM((_L, _P), jnp.float32),
            pltpu.VMEM((_L, _KCAP), jnp.float32),
            pltpu.VMEM((_L, _KCAP), jnp.int32),
        ],
    )(text_features, projected_layer_features, W1_w, W1_b.reshape(1, _RD),
      wcif, wcifb.reshape(1, 3 * _D), score_q_w, score_k_w,
      ln_g.reshape(1, _D), ln_b.reshape(1, _D))


_NW_ROWS = 8


def _gather_rows(table, idx):
    info = plsc.get_sparse_core_info()
    nc = info.num_cores
    mesh = plsc.VectorSubcoreMesh(core_axis_name="c", subcore_axis_name="s")

    @functools.partial(
        pl.kernel,
        mesh=mesh,
        out_type=jax.ShapeDtypeStruct((_FINAL_K, _D), jnp.float32),
        scratch_types=[
            pltpu.VMEM((_NW_ROWS,), jnp.int32),
            pltpu.VMEM((_NW_ROWS, _D), jnp.float32),
            pltpu.SemaphoreType.DMA,
        ],
    )
    def gather_k(table_hbm, idx_hbm, out_hbm, idx_v, rows_v, sem):
        wid = lax.axis_index("s") * nc + lax.axis_index("c")

        @pl.when(wid < _FINAL_K // _NW_ROWS)
        def _():
            base = wid * _NW_ROWS
            pltpu.sync_copy(idx_hbm.at[pl.ds(base, _NW_ROWS)], idx_v)
            pltpu.async_copy(table_hbm.at[idx_v], rows_v, sem).wait()
            pltpu.sync_copy(rows_v, out_hbm.at[pl.ds(base, _NW_ROWS)])

    return gather_k(table, idx)


def kernel(text_features, projected_layer_features, W1_w, W1_b, Wc_w, Wc_b,
           Wi_w, Wi_b, Wf_w, Wf_b, bc, bi, bf, score_q_w, score_k_w, ln_g,
           ln_b):
    sel = _compute_sel(text_features, projected_layer_features, W1_w, W1_b,
                       Wc_w, Wc_b, Wi_w, Wi_b, Wf_w, Wf_b, bc, bi, bf,
                       score_q_w, score_k_w, ln_g, ln_b)
    idx = sel[0]
    table = projected_layer_features.reshape(_L * _P, _D)
    return _gather_rows(table, idx)


# speedup vs baseline: 2.7270x; 1.1474x over previous
import functools

import jax
import jax.numpy as jnp
from jax import lax
from jax.experimental import pallas as pl
from jax.experimental.pallas import tpu as pltpu
from jax.experimental.pallas import tpu_sc as plsc

_T, _D, _L, _P = 2048, 768, 24, 1024
_RD = 192
_FOCUS = 22
_KCAP = 64
_FOCUS_MIN = 32
_FINAL_K = 64
_REST = 64
_THRESH = 2.0
_NEG = float("-inf")


def _ln_rows(x, g, b, eps=1e-5):
    mu = jnp.mean(x, axis=-1, keepdims=True)
    var = jnp.mean((x - mu) ** 2, axis=-1, keepdims=True)
    return (x - mu) / jnp.sqrt(var + eps) * g + b


def _dot_t(a, b):
    return lax.dot_general(a, b, (((1,), (1,)), ((), ())),
                           preferred_element_type=jnp.float32)


_LB = 4
_NB = _L // _LB


def _score_body(text_ref, pl_ref, w1_ref, w1b_ref, wcif_ref, wcifb_ref,
                qw_ref, kw_ref, g_ref, b_ref, sel_ref, q_ref, zm_ref,
                tv_ref, ti_ref):
    i = pl.program_id(0)

    @pl.when(i == 0)
    def _init():
        tm = jnp.mean(text_ref[...], axis=0, keepdims=True)
        q_ref[2:3, :] = _ln_rows(tm, 1.0, 0.0)
        q_ref[1:2, :] = jnp.zeros((1, _D), jnp.float32)

    @pl.when(i < _NB)
    def _rec_steps():
        tg = q_ref[2:3, :]
        c = q_ref[1:2, :]
        for j in range(_LB):
            ym = jnp.mean(pl_ref[j], axis=0, keepdims=True)
            cn = jax.nn.sigmoid(c)
            comb = jnp.concatenate([cn, ym, tg], axis=1)
            s = jnp.maximum(_dot_t(comb, w1_ref[...]) + w1b_ref[0:1, :],
                            0.0)
            u = _dot_t(s, wcif_ref[...]) + wcifb_ref[0:1, :]
            ct = jnp.tanh(u[:, 0:_D])
            ig = jax.nn.sigmoid(u[:, _D:2 * _D])
            fg = jax.nn.sigmoid(u[:, 2 * _D:3 * _D])
            c = fg * c + ig * ct
            if (_L - 2) % _LB == j:
                cq = c

                @pl.when(i == (_L - 2) // _LB)
                def _finalize_q():
                    q_ref[0:1, :] = _ln_rows(_dot_t(cq, qw_ref[...]),
                                             g_ref[0:1, :], b_ref[0:1, :])
        q_ref[1:2, :] = c

    @pl.when(i >= _NB)
    def _scores():
        lbase = (i - _NB) * _LB
        for j in range(_LB):
            kkr = _dot_t(pl_ref[j], kw_ref[...])
            mu = jnp.mean(kkr, axis=1, keepdims=True)
            d = kkr - mu
            var = jnp.mean(d * d, axis=1, keepdims=True)
            inv = 1.0 / jnp.sqrt(var + 1e-5)
            kk = d * inv * g_ref[0:1, :] + b_ref[0:1, :]
            sc = jnp.sum(kk * q_ref[0:1, :], axis=1, keepdims=True)
            m = jnp.mean(sc)
            sd = jnp.sqrt(jnp.mean((sc - m) ** 2))
            z = (sc - m) / (sd + 1e-6)
            zm_ref[pl.ds(lbase + j, 1), :] = z.reshape(1, _P)

    @pl.when(i == 2 * _NB - 1)
    def _select():
        lane_p = lax.broadcasted_iota(jnp.int32, (_L, _P), 1)
        zm = zm_ref[...]
        for k in range(_KCAP):
            mx = jnp.max(zm, axis=1, keepdims=True)
            idx = jnp.min(jnp.where(zm == mx, lane_p, _P), axis=1,
                          keepdims=True)
            tv_ref[:, k:k + 1] = mx
            ti_ref[:, k:k + 1] = idx
            zm = jnp.where(lane_p == idx, _NEG, zm)

        i_col = lax.broadcasted_iota(jnp.int32, (_L, 1), 0)
        conf = tv_ref[:, 0:1]
        mask = (conf > _THRESH) & (i_col != _FOCUS)
        cmax = jnp.max(jnp.where(mask, conf, _NEG), axis=0, keepdims=True)
        e = jnp.where(mask, jnp.exp(conf - cmax), 0.0)
        s_sum = jnp.sum(e, axis=0, keepdims=True)
        w = e / jnp.where(s_sum > 0.0, s_sum, 1.0)
        alloc = jnp.floor(w * float(_REST)).astype(jnp.int32)
        rem = jnp.int32(_REST) - jnp.sum(alloc, axis=0, keepdims=True)
        rank = jnp.zeros((_L, 1), jnp.int32)
        for j in range(_L):
            wj = w[j:j + 1, 0:1]
            rank = rank + jnp.where(wj > w, 1, 0) \
                        + jnp.where((wj == w) & (j < i_col), 1, 0)
        alloc = alloc + jnp.where(mask & (rank < rem), 1, 0)
        kpl = jnp.where(mask, jnp.minimum(alloc, _P), 0)
        kpl = jnp.where(i_col == _FOCUS, _KCAP, kpl)

        col = lax.broadcasted_iota(jnp.int32, (_L, _KCAP), 1)
        row = lax.broadcasted_iota(jnp.int32, (_L, _KCAP), 0)
        cand = jnp.where(col < kpl, tv_ref[...], _NEG)
        rem_sc = jnp.where((row == _FOCUS) & (col < _FOCUS_MIN), _NEG, cand)
        flat = row * _KCAP + col
        tiv = ti_ref[...]

        sel_ref[0:1, 0:_FOCUS_MIN] = (
            jnp.int32(_FOCUS * _P) + ti_ref[_FOCUS:_FOCUS + 1, 0:_FOCUS_MIN])
        for k in range(_FINAL_K - _FOCUS_MIN):
            mx = jnp.max(rem_sc, axis=(0, 1), keepdims=True)
            fidx = jnp.min(jnp.where(rem_sc == mx, flat, _L * _KCAP),
                           axis=(0, 1), keepdims=True)
            onehot = flat == fidx
            pid = jnp.sum(jnp.where(onehot, tiv, 0), axis=(0, 1),
                          keepdims=True)
            lid = fidx // _KCAP
            sel_ref[0:1, _FOCUS_MIN + k:_FOCUS_MIN + k + 1] = (
                lid * _P + pid)
            rem_sc = jnp.where(onehot, _NEG, rem_sc)


def _compute_sel(text_features, projected_layer_features, W1_w, W1_b, Wc_w,
                 Wc_b, Wi_w, Wi_b, Wf_w, Wf_b, bc, bi, bf, score_q_w,
                 score_k_w, ln_g, ln_b):
    def c0(i):
        return (0, 0)
    wcif = jnp.concatenate([Wc_w, Wi_w, Wf_w], axis=0)
    wcifb = jnp.concatenate([Wc_b + bc, Wi_b + bi, Wf_b + bf])
    return pl.pallas_call(
        _score_body,
        grid=(2 * _NB,),
        in_specs=[
            pl.BlockSpec((_T, _D), c0),
            pl.BlockSpec((_LB, _P, _D),
                         lambda i: (jnp.where(i < _NB, i, i - _NB), 0, 0)),
            pl.BlockSpec((_RD, 3 * _D), c0),
            pl.BlockSpec((1, _RD), c0),
            pl.BlockSpec((3 * _D, _RD), c0),
            pl.BlockSpec((1, 3 * _D), c0),
            pl.BlockSpec((_D, _D), c0),
            pl.BlockSpec((_D, _D), c0),
            pl.BlockSpec((1, _D), c0),
            pl.BlockSpec((1, _D), c0),
        ],
        out_specs=pl.BlockSpec((8, _FINAL_K), c0),
        out_shape=jax.ShapeDtypeStruct((8, _FINAL_K), jnp.int32),
        scratch_shapes=[
            pltpu.VMEM((8, _D), jnp.float32),
            pltpu.VMEM((_L, _P), jnp.float32),
            pltpu.VMEM((_L, _KCAP), jnp.float32),
            pltpu.VMEM((_L, _KCAP), jnp.int32),
        ],
    )(text_features, projected_layer_features, W1_w, W1_b.reshape(1, _RD),
      wcif, wcifb.reshape(1, 3 * _D), score_q_w, score_k_w,
      ln_g.reshape(1, _D), ln_b.reshape(1, _D))


_NW_ROWS = 8


def _gather_rows(table, idx):
    info = plsc.get_sparse_core_info()
    nc = info.num_cores
    mesh = plsc.VectorSubcoreMesh(core_axis_name="c", subcore_axis_name="s")

    @functools.partial(
        pl.kernel,
        mesh=mesh,
        out_type=jax.ShapeDtypeStruct((_FINAL_K, _D), jnp.float32),
        scratch_types=[
            pltpu.VMEM((_NW_ROWS,), jnp.int32),
            pltpu.VMEM((_NW_ROWS, _D), jnp.float32),
            pltpu.SemaphoreType.DMA,
        ],
    )
    def gather_k(table_hbm, idx_hbm, out_hbm, idx_v, rows_v, sem):
        wid = lax.axis_index("s") * nc + lax.axis_index("c")

        @pl.when(wid < _FINAL_K // _NW_ROWS)
        def _():
            base = wid * _NW_ROWS
            pltpu.sync_copy(idx_hbm.at[pl.ds(base, _NW_ROWS)], idx_v)
            pltpu.async_copy(table_hbm.at[idx_v], rows_v, sem).wait()
            pltpu.sync_copy(rows_v, out_hbm.at[pl.ds(base, _NW_ROWS)])

    return gather_k(table, idx)


def kernel(text_features, projected_layer_features, W1_w, W1_b, Wc_w, Wc_b,
           Wi_w, Wi_b, Wf_w, Wf_b, bc, bi, bf, score_q_w, score_k_w, ln_g,
           ln_b):
    sel = _compute_sel(text_features, projected_layer_features, W1_w, W1_b,
                       Wc_w, Wc_b, Wi_w, Wi_b, Wf_w, Wf_b, bc, bi, bf,
                       score_q_w, score_k_w, ln_g, ln_b)
    idx = sel[0]
    table = projected_layer_features.reshape(_L * _P, _D)
    return _gather_rows(table, idx)


# speedup vs baseline: 2.7522x; 1.0093x over previous
import functools

import jax
import jax.numpy as jnp
from jax import lax
from jax.experimental import pallas as pl
from jax.experimental.pallas import tpu as pltpu
from jax.experimental.pallas import tpu_sc as plsc

_T, _D, _L, _P = 2048, 768, 24, 1024
_RD = 192
_FOCUS = 22
_KCAP = 64
_FOCUS_MIN = 32
_FINAL_K = 64
_REST = 64
_THRESH = 2.0
_NEG = float("-inf")


def _ln_rows(x, g, b, eps=1e-5):
    mu = jnp.mean(x, axis=-1, keepdims=True)
    var = jnp.mean((x - mu) ** 2, axis=-1, keepdims=True)
    return (x - mu) / jnp.sqrt(var + eps) * g + b


def _dot_t(a, b):
    return lax.dot_general(a, b, (((1,), (1,)), ((), ())),
                           preferred_element_type=jnp.float32)


_LB = 6
_NB = _L // _LB


def _score_body(text_ref, pl_ref, w1_ref, w1b_ref, wcif_ref, wcifb_ref,
                qw_ref, kw_ref, g_ref, b_ref, sel_ref, q_ref, zm_ref,
                tv_ref, ti_ref):
    i = pl.program_id(0)

    @pl.when(i == 0)
    def _init():
        tm = jnp.mean(text_ref[...], axis=0, keepdims=True)
        q_ref[2:3, :] = _ln_rows(tm, 1.0, 0.0)
        q_ref[1:2, :] = jnp.zeros((1, _D), jnp.float32)

    @pl.when(i < _NB)
    def _rec_steps():
        tg = q_ref[2:3, :]
        c = q_ref[1:2, :]
        for j in range(_LB):
            ym = jnp.mean(pl_ref[j], axis=0, keepdims=True)
            cn = jax.nn.sigmoid(c)
            comb = jnp.concatenate([cn, ym, tg], axis=1)
            s = jnp.maximum(_dot_t(comb, w1_ref[...]) + w1b_ref[0:1, :],
                            0.0)
            u = _dot_t(s, wcif_ref[...]) + wcifb_ref[0:1, :]
            ct = jnp.tanh(u[:, 0:_D])
            ig = jax.nn.sigmoid(u[:, _D:2 * _D])
            fg = jax.nn.sigmoid(u[:, 2 * _D:3 * _D])
            c = fg * c + ig * ct
            if (_L - 2) % _LB == j:
                cq = c

                @pl.when(i == (_L - 2) // _LB)
                def _finalize_q():
                    q_ref[0:1, :] = _ln_rows(_dot_t(cq, qw_ref[...]),
                                             g_ref[0:1, :], b_ref[0:1, :])
        q_ref[1:2, :] = c

    @pl.when(i >= _NB)
    def _scores():
        lbase = (i - _NB) * _LB
        for j in range(_LB):
            kkr = _dot_t(pl_ref[j], kw_ref[...])
            mu = jnp.mean(kkr, axis=1, keepdims=True)
            d = kkr - mu
            var = jnp.mean(d * d, axis=1, keepdims=True)
            inv = 1.0 / jnp.sqrt(var + 1e-5)
            kk = d * inv * g_ref[0:1, :] + b_ref[0:1, :]
            sc = jnp.sum(kk * q_ref[0:1, :], axis=1, keepdims=True)
            m = jnp.mean(sc)
            sd = jnp.sqrt(jnp.mean((sc - m) ** 2))
            z = (sc - m) / (sd + 1e-6)
            zm_ref[pl.ds(lbase + j, 1), :] = z.reshape(1, _P)

    @pl.when(i == 2 * _NB - 1)
    def _select():
        lane_p = lax.broadcasted_iota(jnp.int32, (_L, _P), 1)
        zm = zm_ref[...]
        for k in range(_KCAP):
            mx = jnp.max(zm, axis=1, keepdims=True)
            idx = jnp.min(jnp.where(zm == mx, lane_p, _P), axis=1,
                          keepdims=True)
            tv_ref[:, k:k + 1] = mx
            ti_ref[:, k:k + 1] = idx
            zm = jnp.where(lane_p == idx, _NEG, zm)

        i_col = lax.broadcasted_iota(jnp.int32, (_L, 1), 0)
        conf = tv_ref[:, 0:1]
        mask = (conf > _THRESH) & (i_col != _FOCUS)
        cmax = jnp.max(jnp.where(mask, conf, _NEG), axis=0, keepdims=True)
        e = jnp.where(mask, jnp.exp(conf - cmax), 0.0)
        s_sum = jnp.sum(e, axis=0, keepdims=True)
        w = e / jnp.where(s_sum > 0.0, s_sum, 1.0)
        alloc = jnp.floor(w * float(_REST)).astype(jnp.int32)
        rem = jnp.int32(_REST) - jnp.sum(alloc, axis=0, keepdims=True)
        rank = jnp.zeros((_L, 1), jnp.int32)
        for j in range(_L):
            wj = w[j:j + 1, 0:1]
            rank = rank + jnp.where(wj > w, 1, 0) \
                        + jnp.where((wj == w) & (j < i_col), 1, 0)
        alloc = alloc + jnp.where(mask & (rank < rem), 1, 0)
        kpl = jnp.where(mask, jnp.minimum(alloc, _P), 0)
        kpl = jnp.where(i_col == _FOCUS, _KCAP, kpl)

        col = lax.broadcasted_iota(jnp.int32, (_L, _KCAP), 1)
        row = lax.broadcasted_iota(jnp.int32, (_L, _KCAP), 0)
        cand = jnp.where(col < kpl, tv_ref[...], _NEG)
        rem_sc = jnp.where((row == _FOCUS) & (col < _FOCUS_MIN), _NEG, cand)
        flat = row * _KCAP + col
        tiv = ti_ref[...]

        sel_ref[0:1, 0:_FOCUS_MIN] = (
            jnp.int32(_FOCUS * _P) + ti_ref[_FOCUS:_FOCUS + 1, 0:_FOCUS_MIN])
        for k in range(_FINAL_K - _FOCUS_MIN):
            mx = jnp.max(rem_sc, axis=(0, 1), keepdims=True)
            fidx = jnp.min(jnp.where(rem_sc == mx, flat, _L * _KCAP),
                           axis=(0, 1), keepdims=True)
            onehot = flat == fidx
            pid = jnp.sum(jnp.where(onehot, tiv, 0), axis=(0, 1),
                          keepdims=True)
            lid = fidx // _KCAP
            sel_ref[0:1, _FOCUS_MIN + k:_FOCUS_MIN + k + 1] = (
                lid * _P + pid)
            rem_sc = jnp.where(onehot, _NEG, rem_sc)


def _compute_sel(text_features, projected_layer_features, W1_w, W1_b, Wc_w,
                 Wc_b, Wi_w, Wi_b, Wf_w, Wf_b, bc, bi, bf, score_q_w,
                 score_k_w, ln_g, ln_b):
    def c0(i):
        return (0, 0)
    wcif = jnp.concatenate([Wc_w, Wi_w, Wf_w], axis=0)
    wcifb = jnp.concatenate([Wc_b + bc, Wi_b + bi, Wf_b + bf])
    return pl.pallas_call(
        _score_body,
        grid=(2 * _NB,),
        in_specs=[
            pl.BlockSpec((_T, _D), c0),
            pl.BlockSpec((_LB, _P, _D),
                         lambda i: (jnp.where(i < _NB, i, i - _NB), 0, 0)),
            pl.BlockSpec((_RD, 3 * _D), c0),
            pl.BlockSpec((1, _RD), c0),
            pl.BlockSpec((3 * _D, _RD), c0),
            pl.BlockSpec((1, 3 * _D), c0),
            pl.BlockSpec((_D, _D), c0),
            pl.BlockSpec((_D, _D), c0),
            pl.BlockSpec((1, _D), c0),
            pl.BlockSpec((1, _D), c0),
        ],
        out_specs=pl.BlockSpec((8, _FINAL_K), c0),
        out_shape=jax.ShapeDtypeStruct((8, _FINAL_K), jnp.int32),
        scratch_shapes=[
            pltpu.VMEM((8, _D), jnp.float32),
            pltpu.VMEM((_L, _P), jnp.float32),
            pltpu.VMEM((_L, _KCAP), jnp.float32),
            pltpu.VMEM((_L, _KCAP), jnp.int32),
        ],
    )(text_features, projected_layer_features, W1_w, W1_b.reshape(1, _RD),
      wcif, wcifb.reshape(1, 3 * _D), score_q_w, score_k_w,
      ln_g.reshape(1, _D), ln_b.reshape(1, _D))


_NW_ROWS = 8


def _gather_rows(table, idx):
    info = plsc.get_sparse_core_info()
    nc = info.num_cores
    mesh = plsc.VectorSubcoreMesh(core_axis_name="c", subcore_axis_name="s")

    @functools.partial(
        pl.kernel,
        mesh=mesh,
        out_type=jax.ShapeDtypeStruct((_FINAL_K, _D), jnp.float32),
        scratch_types=[
            pltpu.VMEM((_NW_ROWS,), jnp.int32),
            pltpu.VMEM((_NW_ROWS, _D), jnp.float32),
            pltpu.SemaphoreType.DMA,
        ],
    )
    def gather_k(table_hbm, idx_hbm, out_hbm, idx_v, rows_v, sem):
        wid = lax.axis_index("s") * nc + lax.axis_index("c")

        @pl.when(wid < _FINAL_K // _NW_ROWS)
        def _():
            base = wid * _NW_ROWS
            pltpu.sync_copy(idx_hbm.at[pl.ds(base, _NW_ROWS)], idx_v)
            pltpu.async_copy(table_hbm.at[idx_v], rows_v, sem).wait()
            pltpu.sync_copy(rows_v, out_hbm.at[pl.ds(base, _NW_ROWS)])

    return gather_k(table, idx)


def kernel(text_features, projected_layer_features, W1_w, W1_b, Wc_w, Wc_b,
           Wi_w, Wi_b, Wf_w, Wf_b, bc, bi, bf, score_q_w, score_k_w, ln_g,
           ln_b):
    sel = _compute_sel(text_features, projected_layer_features, W1_w, W1_b,
                       Wc_w, Wc_b, Wi_w, Wi_b, Wf_w, Wf_b, bc, bi, bf,
                       score_q_w, score_k_w, ln_g, ln_b)
    idx = sel[0]
    table = projected_layer_features.reshape(_L * _P, _D)
    return _gather_rows(table, idx)
